# Initial kernel scaffold; baseline (speedup 1.0000x reference)
import jax
import jax.numpy as jnp
from jax.experimental import pallas as pl


def kernel(x, y, normalizer):
    raise NotImplementedError("write your pallas kernel here")


# speedup vs baseline: 2.5587x; 2.5587x over previous
import functools
import math

import jax
import jax.numpy as jnp
from jax import lax
from jax.experimental import pallas as pl
from jax.experimental.pallas import tpu as pltpu
from jax.experimental.pallas import tpu_sc as plsc

VOCAB = 32000
SMOOTH = 0.1
CONF = 1.0 - SMOOTH
EPS = SMOOTH / (VOCAB - 2)
ROW_CONST = float((VOCAB - 2) * EPS * math.log(EPS) + CONF * math.log(CONF))

LANES = 16
NUM_WORKERS = 32


def _sc_gather_body(xf_hbm, y_hbm, out_hbm, y_v, idxg_v, idx0_v, vals_g,
                    vals_0, acc_v, sem_g, sem_0, *, rows_per_worker):
    wid = lax.axis_index("s") * 2 + lax.axis_index("c")
    base = wid * rows_per_worker
    nchunks = rows_per_worker // LANES

    pltpu.sync_copy(y_hbm.at[pl.ds(base, rows_per_worker)], y_v)

    iota = lax.iota(jnp.int32, LANES)
    for j in range(nchunks):
        yv = y_v[pl.ds(j * LANES, LANES)]
        rowid = base + j * LANES + iota
        idxg_v[pl.ds(j * LANES, LANES)] = rowid * VOCAB + yv
        idx0_v[pl.ds(j * LANES, LANES)] = rowid * VOCAB

    cp_g = pltpu.async_copy(xf_hbm.at[idxg_v], vals_g, sem_g)
    cp_0 = pltpu.async_copy(xf_hbm.at[idx0_v], vals_0, sem_0)
    cp_g.wait()
    cp_0.wait()

    zeros_f = jnp.zeros((LANES,), jnp.float32)
    acc = zeros_f
    for j in range(nchunks):
        yv = y_v[pl.ds(j * LANES, LANES)]
        g = vals_g[pl.ds(j * LANES, LANES)]
        x0 = vals_0[pl.ds(j * LANES, LANES)]
        val = ROW_CONST + EPS * x0 + (EPS - CONF) * g
        acc = acc + jnp.where(yv != 0, val, zeros_f)

    acc_v[pl.ds(0, LANES)] = acc
    for t in range(1, 128 // LANES):
        acc_v[pl.ds(t * LANES, LANES)] = zeros_f
    pltpu.sync_copy(acc_v, out_hbm.at[wid])


def _sc_gather(xf, y32, rows_per_worker):
    mesh = plsc.VectorSubcoreMesh(core_axis_name="c", subcore_axis_name="s",
                                  num_cores=2, num_subcores=16)
    kern = pl.kernel(
        functools.partial(_sc_gather_body, rows_per_worker=rows_per_worker),
        out_type=jax.ShapeDtypeStruct((NUM_WORKERS, 128), jnp.float32),
        mesh=mesh,
        scratch_types=[
            pltpu.VMEM((rows_per_worker,), jnp.int32),
            pltpu.VMEM((rows_per_worker,), jnp.int32),
            pltpu.VMEM((rows_per_worker,), jnp.int32),
            pltpu.VMEM((rows_per_worker,), jnp.float32),
            pltpu.VMEM((rows_per_worker,), jnp.float32),
            pltpu.VMEM((128,), jnp.float32),
            pltpu.SemaphoreType.DMA,
            pltpu.SemaphoreType.DMA,
        ],
    )
    return kern(xf, y32)


def _tc_body(x_ref, y_ref, sc_ref, norm_ref, o_ref, acc_ref):
    i = pl.program_id(0)
    n = pl.num_programs(0)

    @pl.when(i == 0)
    def _():
        acc_ref[0, 0] = 0.0

    row_sums = jnp.sum(x_ref[...], axis=1)
    mask = y_ref[0, 0, :] != 0
    acc_ref[0, 0] += jnp.sum(jnp.where(mask, row_sums, 0.0))

    @pl.when(i == n - 1)
    def _():
        o_ref[0, 0] = (jnp.sum(sc_ref[...])
                       - EPS * acc_ref[0, 0]) / norm_ref[0, 0]


def kernel(x, y, normalizer):
    n, vocab = x.shape
    y32 = y.astype(jnp.int32)
    rows_per_worker = n // NUM_WORKERS

    xf = x.reshape(n * vocab)
    sc_part = _sc_gather(xf, y32, rows_per_worker)

    row_blk = 64
    grid = n // row_blk
    y3 = y32.reshape(grid, 1, row_blk)
    norm = jnp.asarray(normalizer, jnp.float32).reshape(1, 1)

    out = pl.pallas_call(
        _tc_body,
        grid=(grid,),
        in_specs=[
            pl.BlockSpec((row_blk, vocab), lambda i: (i, 0)),
            pl.BlockSpec((1, 1, row_blk), lambda i: (i, 0, 0)),
            pl.BlockSpec((NUM_WORKERS, 128), lambda i: (0, 0)),
            pl.BlockSpec(memory_space=pltpu.SMEM),
        ],
        out_specs=pl.BlockSpec(memory_space=pltpu.SMEM),
        out_shape=jax.ShapeDtypeStruct((1, 1), jnp.float32),
        scratch_shapes=[pltpu.SMEM((1, 1), jnp.float32)],
    )(x, y3, sc_part, norm)
    return out[0, 0]


# speedup vs baseline: 2.5630x; 1.0017x over previous
import functools
import math

import jax
import jax.numpy as jnp
from jax import lax
from jax.experimental import pallas as pl
from jax.experimental.pallas import tpu as pltpu
from jax.experimental.pallas import tpu_sc as plsc

VOCAB = 32000
SMOOTH = 0.1
CONF = 1.0 - SMOOTH
EPS = SMOOTH / (VOCAB - 2)
ROW_CONST = float((VOCAB - 2) * EPS * math.log(EPS) + CONF * math.log(CONF))

LANES = 16
NUM_WORKERS = 32


def _sc_gather_body(xf_hbm, y_hbm, out_hbm, y_v, idxg_v, idx0_v, vals_g,
                    vals_0, acc_v, sem_g, sem_0, *, rows_per_worker):
    wid = lax.axis_index("s") * 2 + lax.axis_index("c")
    base = wid * rows_per_worker
    nchunks = rows_per_worker // LANES

    pltpu.sync_copy(y_hbm.at[pl.ds(base, rows_per_worker)], y_v)

    iota = lax.iota(jnp.int32, LANES)
    for j in range(nchunks):
        yv = y_v[pl.ds(j * LANES, LANES)]
        rowid = base + j * LANES + iota
        idxg_v[pl.ds(j * LANES, LANES)] = rowid * VOCAB + yv
        idx0_v[pl.ds(j * LANES, LANES)] = rowid * VOCAB

    cp_g = pltpu.async_copy(xf_hbm.at[idxg_v], vals_g, sem_g)
    cp_0 = pltpu.async_copy(xf_hbm.at[idx0_v], vals_0, sem_0)
    cp_g.wait()
    cp_0.wait()

    zeros_f = jnp.zeros((LANES,), jnp.float32)
    acc = zeros_f
    for j in range(nchunks):
        yv = y_v[pl.ds(j * LANES, LANES)]
        g = vals_g[pl.ds(j * LANES, LANES)]
        x0 = vals_0[pl.ds(j * LANES, LANES)]
        val = ROW_CONST + EPS * x0 + (EPS - CONF) * g
        acc = acc + jnp.where(yv != 0, val, zeros_f)

    acc_v[pl.ds(0, LANES)] = acc
    for t in range(1, 128 // LANES):
        acc_v[pl.ds(t * LANES, LANES)] = zeros_f
    pltpu.sync_copy(acc_v, out_hbm.at[wid])


def _sc_gather(xf, y32, rows_per_worker):
    mesh = plsc.VectorSubcoreMesh(core_axis_name="c", subcore_axis_name="s",
                                  num_cores=2, num_subcores=16)
    kern = pl.kernel(
        functools.partial(_sc_gather_body, rows_per_worker=rows_per_worker),
        out_type=jax.ShapeDtypeStruct((NUM_WORKERS, 128), jnp.float32),
        mesh=mesh,
        scratch_types=[
            pltpu.VMEM((rows_per_worker,), jnp.int32),
            pltpu.VMEM((rows_per_worker,), jnp.int32),
            pltpu.VMEM((rows_per_worker,), jnp.int32),
            pltpu.VMEM((rows_per_worker,), jnp.float32),
            pltpu.VMEM((rows_per_worker,), jnp.float32),
            pltpu.VMEM((128,), jnp.float32),
            pltpu.SemaphoreType.DMA,
            pltpu.SemaphoreType.DMA,
        ],
    )
    return kern(xf, y32)


def _tc_body(x_ref, y_ref, sc_ref, norm_ref, o_ref, acc_ref):
    i = pl.program_id(0)
    n = pl.num_programs(0)

    @pl.when(i == 0)
    def _():
        acc_ref[0, 0] = 0.0

    row_sums = jnp.sum(x_ref[...], axis=1)
    mask = y_ref[0, 0, :] != 0
    acc_ref[0, 0] += jnp.sum(jnp.where(mask, row_sums, 0.0))

    @pl.when(i == n - 1)
    def _():
        o_ref[0, 0] = (jnp.sum(sc_ref[...])
                       - EPS * acc_ref[0, 0]) / norm_ref[0, 0]


def kernel(x, y, normalizer):
    n, vocab = x.shape
    y32 = y.astype(jnp.int32)
    rows_per_worker = n // NUM_WORKERS

    xf = x.reshape(n * vocab)
    sc_part = _sc_gather(xf, y32, rows_per_worker)

    row_blk = 128
    grid = n // row_blk
    y3 = y32.reshape(grid, 1, row_blk)
    norm = jnp.asarray(normalizer, jnp.float32).reshape(1, 1)

    out = pl.pallas_call(
        _tc_body,
        grid=(grid,),
        in_specs=[
            pl.BlockSpec((row_blk, vocab), lambda i: (i, 0)),
            pl.BlockSpec((1, 1, row_blk), lambda i: (i, 0, 0)),
            pl.BlockSpec((NUM_WORKERS, 128), lambda i: (0, 0)),
            pl.BlockSpec(memory_space=pltpu.SMEM),
        ],
        out_specs=pl.BlockSpec(memory_space=pltpu.SMEM),
        out_shape=jax.ShapeDtypeStruct((1, 1), jnp.float32),
        scratch_shapes=[pltpu.SMEM((1, 1), jnp.float32)],
    )(x, y3, sc_part, norm)
    return out[0, 0]
